# baseline (speedup 1.0000x reference)
import functools

import jax
import jax.numpy as jnp
from jax import lax
from jax.experimental import pallas as pl
from jax.experimental.pallas import tpu as pltpu
from jax.experimental.pallas import tpu_sc as plsc

PATCH_SIZE = 16
SEARCH_SIZE = 384

_NUM_TILES = 32
_LANES = 16


def _make_sc_reduce(rows_total, num_chunks):
    rows_per_tile = rows_total // _NUM_TILES
    mesh = plsc.VectorSubcoreMesh(core_axis_name="c", subcore_axis_name="s")

    @functools.partial(
        pl.kernel,
        mesh=mesh,
        out_type=jax.ShapeDtypeStruct((_NUM_TILES, _LANES), jnp.float32),
        compiler_params=pltpu.CompilerParams(use_tc_tiling_on_sc=False),
        scratch_types=[
            pltpu.VMEM((num_chunks, _LANES), jnp.float32),
            pltpu.VMEM((num_chunks, _LANES), jnp.int32),
            pltpu.VMEM((rows_per_tile, _LANES), jnp.float32),
            pltpu.VMEM((_LANES,), jnp.float32),
        ],
    )
    def sc_reduce(table_hbm, mask_hbm, flags_hbm, out_hbm, mask_v, flags_v,
                  rows_v, acc_v):
        wid = lax.axis_index("s") * 2 + lax.axis_index("c")
        base_row = wid * rows_per_tile
        pltpu.sync_copy(mask_hbm, mask_v)
        pltpu.sync_copy(flags_hbm, flags_v)
        acc_v[...] = jnp.zeros((_LANES,), jnp.float32)

        def chunk_body(c, carry):
            m = mask_v[c, :]

            @pl.when(flags_v[c, :][0] > 0)
            def _():
                pltpu.sync_copy(
                    table_hbm.at[pl.ds(base_row, rows_per_tile), c], rows_v
                )

                def red_body(i, accs):
                    a0, a1, a2, a3 = accs
                    b = i * 4
                    return (
                        a0 + rows_v[b, :],
                        a1 + rows_v[b + 1, :],
                        a2 + rows_v[b + 2, :],
                        a3 + rows_v[b + 3, :],
                    )

                z = jnp.zeros((_LANES,), jnp.float32)
                a0, a1, a2, a3 = lax.fori_loop(
                    0, rows_per_tile // 4, red_body, (z, z, z, z)
                )
                csum = (a0 + a1) + (a2 + a3)
                acc_v[...] = acc_v[...] + csum * m

            return carry

        lax.fori_loop(0, num_chunks, chunk_body, 0)
        pltpu.sync_copy(acc_v, out_hbm.at[wid])

    return sc_reduce


def kernel(attn_weights, bbox):
    x, y, w, h = bbox[0], bbox[1], bbox[2], bbox[3]
    num_patches = SEARCH_SIZE // PATCH_SIZE
    search_seq_len = num_patches**2
    p = jnp.arange(search_seq_len, dtype=jnp.int32)
    pi = p // num_patches
    pj = p % num_patches
    i_lo = jnp.maximum(0, y // PATCH_SIZE)
    i_hi = jnp.minimum(num_patches, (y + h + PATCH_SIZE - 1) // PATCH_SIZE)
    j_lo = jnp.maximum(0, x // PATCH_SIZE)
    j_hi = jnp.minimum(num_patches, (x + w + PATCH_SIZE - 1) // PATCH_SIZE)
    mask = ((pi >= i_lo) & (pi < i_hi) & (pj >= j_lo) & (pj < j_hi)).astype(
        jnp.float32
    )
    count = mask.sum()
    safe_count = jnp.maximum(count, jnp.float32(1.0))

    num_layers, batch, heads, seq, seq2 = attn_weights.shape
    assert seq == search_seq_len and seq2 == search_seq_len
    rows_total = num_layers * batch * heads * seq
    num_chunks = search_seq_len // _LANES

    table = attn_weights.reshape(rows_total, num_chunks, _LANES)
    mask_chunks = mask.reshape(num_chunks, _LANES)
    flags = jnp.broadcast_to(
        (mask_chunks.sum(axis=1) > 0).astype(jnp.int32)[:, None],
        (num_chunks, _LANES),
    )

    partials = _make_sc_reduce(rows_total, num_chunks)(table, mask_chunks, flags)
    total = jnp.sum(partials)
    loss = total / (safe_count * jnp.float32(rows_total))
    return jnp.where(count == 0, jnp.float32(0.0), loss)


# speedup vs baseline: 3.9334x; 3.9334x over previous
import functools

import jax
import jax.numpy as jnp
from jax import lax
from jax.experimental import pallas as pl
from jax.experimental.pallas import tpu as pltpu
from jax.experimental.pallas import tpu_sc as plsc

PATCH_SIZE = 16
SEARCH_SIZE = 384

_NUM_TILES = 32
_LANES = 16


def _make_sc_reduce(num_mats, seq, num_chunks):
    mats_per_tile = num_mats // _NUM_TILES
    mesh = plsc.VectorSubcoreMesh(core_axis_name="c", subcore_axis_name="s")

    @functools.partial(
        pl.kernel,
        mesh=mesh,
        out_type=jax.ShapeDtypeStruct((_NUM_TILES, _LANES), jnp.float32),
        scratch_types=[
            pltpu.VMEM((num_chunks, _LANES), jnp.float32),
            pltpu.VMEM((num_chunks, _LANES), jnp.int32),
            pltpu.VMEM((seq, 128), jnp.float32),
            pltpu.VMEM((_LANES,), jnp.float32),
        ],
    )
    def sc_reduce(attn_hbm, mask_hbm, flags_hbm, out_hbm, mask_v, flags_v,
                  slab_v, acc_v):
        wid = lax.axis_index("s") * 2 + lax.axis_index("c")
        base_mat = wid * mats_per_tile
        pltpu.sync_copy(mask_hbm, mask_v)
        pltpu.sync_copy(flags_hbm, flags_v)
        acc_v[...] = jnp.zeros((_LANES,), jnp.float32)

        def reduce_slab(off):
            def red_body(i, accs):
                a0, a1, a2, a3 = accs
                b = i * 4
                return (
                    a0 + slab_v[b, pl.ds(off, _LANES)],
                    a1 + slab_v[b + 1, pl.ds(off, _LANES)],
                    a2 + slab_v[b + 2, pl.ds(off, _LANES)],
                    a3 + slab_v[b + 3, pl.ds(off, _LANES)],
                )

            z = jnp.zeros((_LANES,), jnp.float32)
            a0, a1, a2, a3 = lax.fori_loop(0, seq // 4, red_body, (z, z, z, z))
            return (a0 + a1) + (a2 + a3)

        def chunk_body(c, carry):
            m = mask_v[c, :]
            sub = lax.rem(c, 8)
            jtile = lax.div(c, 8)

            @pl.when(flags_v[c, :][0] > 0)
            def _():
                off = sub * _LANES
                col0 = pl.multiple_of(jtile * 128, 128)

                def mat_body(mi, csum):
                    pltpu.sync_copy(
                        attn_hbm.at[base_mat + mi, :, pl.ds(col0, 128)],
                        slab_v,
                    )
                    return csum + reduce_slab(off)

                csum = lax.fori_loop(
                    0, mats_per_tile, mat_body,
                    jnp.zeros((_LANES,), jnp.float32),
                )
                acc_v[...] = acc_v[...] + csum * m

            return carry

        lax.fori_loop(0, (seq // 128) * 8, chunk_body, 0)
        pltpu.sync_copy(acc_v, out_hbm.at[wid])

    return sc_reduce


def kernel(attn_weights, bbox):
    x, y, w, h = bbox[0], bbox[1], bbox[2], bbox[3]
    num_patches = SEARCH_SIZE // PATCH_SIZE
    search_seq_len = num_patches**2
    p = jnp.arange(search_seq_len, dtype=jnp.int32)
    pi = p // num_patches
    pj = p % num_patches
    i_lo = jnp.maximum(0, y // PATCH_SIZE)
    i_hi = jnp.minimum(num_patches, (y + h + PATCH_SIZE - 1) // PATCH_SIZE)
    j_lo = jnp.maximum(0, x // PATCH_SIZE)
    j_hi = jnp.minimum(num_patches, (x + w + PATCH_SIZE - 1) // PATCH_SIZE)
    mask = ((pi >= i_lo) & (pi < i_hi) & (pj >= j_lo) & (pj < j_hi)).astype(
        jnp.float32
    )
    count = mask.sum()
    safe_count = jnp.maximum(count, jnp.float32(1.0))

    num_layers, batch, heads, seq, seq2 = attn_weights.shape
    assert seq == search_seq_len and seq2 == search_seq_len
    num_mats = num_layers * batch * heads
    rows_total = num_mats * seq
    num_chunks = search_seq_len // _LANES

    attn3 = attn_weights.reshape(num_mats, seq, seq)
    mask_chunks = mask.reshape(num_chunks, _LANES)
    flags = jnp.broadcast_to(
        (mask_chunks.sum(axis=1) > 0).astype(jnp.int32)[:, None],
        (num_chunks, _LANES),
    )

    partials = _make_sc_reduce(num_mats, seq, num_chunks)(
        attn3, mask_chunks, flags
    )
    total = jnp.sum(partials)

    tail0 = (seq // 128) * 128
    tail_mask = mask[tail0:]

    def _tail_sum(_):
        return jnp.einsum(
            "mrk,k->", attn3[:, :, tail0:], tail_mask,
            preferred_element_type=jnp.float32,
        )

    tail_total = lax.cond(
        jnp.any(tail_mask > 0), _tail_sum, lambda _: jnp.float32(0.0), 0
    )

    loss = (total + tail_total) / (safe_count * jnp.float32(rows_total))
    return jnp.where(count == 0, jnp.float32(0.0), loss)


# speedup vs baseline: 4.6811x; 1.1901x over previous
import functools

import jax
import jax.numpy as jnp
from jax import lax
from jax.experimental import pallas as pl
from jax.experimental.pallas import tpu as pltpu
from jax.experimental.pallas import tpu_sc as plsc

PATCH_SIZE = 16
SEARCH_SIZE = 384

_NUM_TILES = 32
_LANES = 16
_HALF = 288


def _make_sc_reduce(num_mats, seq, num_patches):
    mats_per_tile = num_mats // _NUM_TILES
    full_chunks = (seq // 128) * 8
    pieces = mats_per_tile * 2
    mesh = plsc.VectorSubcoreMesh(core_axis_name="c", subcore_axis_name="s")

    @functools.partial(
        pl.kernel,
        mesh=mesh,
        out_type=[
            jax.ShapeDtypeStruct((_NUM_TILES, _LANES), jnp.float32),
            jax.ShapeDtypeStruct((_LANES,), jnp.float32),
        ],
        scratch_types=[
            pltpu.VMEM((_LANES,), jnp.int32),
            pltpu.VMEM((_HALF, 128), jnp.float32),
            pltpu.VMEM((_HALF, 128), jnp.float32),
            pltpu.VMEM((_LANES,), jnp.float32),
            pltpu.SemaphoreType.DMA,
            pltpu.SemaphoreType.DMA,
        ],
    )
    def sc_reduce(attn_hbm, bbox_hbm, out_hbm, cnt_hbm, bbox_v, slab_a,
                  slab_b, acc_v, sem_a, sem_b):
        wid = lax.axis_index("s") * 2 + lax.axis_index("c")
        base_mat = wid * mats_per_tile
        pltpu.sync_copy(bbox_hbm, bbox_v)
        bb = bbox_v[...]
        x, y, w, h = bb[0], bb[1], bb[2], bb[3]
        i_lo = jnp.maximum(0, y >> 4)
        i_hi = jnp.minimum(num_patches, (y + h + PATCH_SIZE - 1) >> 4)
        j_lo = jnp.maximum(0, x >> 4)
        j_hi = jnp.minimum(num_patches, (x + w + PATCH_SIZE - 1) >> 4)
        lane = jax.lax.iota(jnp.int32, _LANES)

        slabs = (slab_a, slab_b)
        sems = (sem_a, sem_b)

        def piece_src(c_jtile, j):
            mat = base_mat + j // 2
            r0 = (j % 2) * _HALF
            col0 = pl.multiple_of(c_jtile * 128, 128)
            return attn_hbm.at[mat, pl.ds(r0, _HALF), pl.ds(col0, 128)]

        def reduce_half(buf, off):
            def red_body(i, accs):
                a0, a1, a2, a3 = accs
                b = i * 4
                return (
                    a0 + buf[b, pl.ds(off, _LANES)],
                    a1 + buf[b + 1, pl.ds(off, _LANES)],
                    a2 + buf[b + 2, pl.ds(off, _LANES)],
                    a3 + buf[b + 3, pl.ds(off, _LANES)],
                )

            z = jnp.zeros((_LANES,), jnp.float32)
            a0, a1, a2, a3 = lax.fori_loop(0, _HALF // 4, red_body,
                                           (z, z, z, z))
            return (a0 + a1) + (a2 + a3)

        def seg_hit(ai, rlo, rhi):
            row_ok = jnp.logical_and(ai >= i_lo, ai < i_hi)
            seg_ok = jnp.maximum(rlo, j_lo) < jnp.minimum(rhi, j_hi)
            return jnp.logical_and(row_ok, seg_ok)

        acc_v[...] = jnp.zeros((_LANES,), jnp.float32)

        def chunk_body(c, carry):
            pi, pj, a_i, a_r = carry
            m_bool = (pi >= i_lo) & (pi < i_hi) & (pj >= j_lo) & (pj < j_hi)
            m = jnp.where(
                m_bool,
                jnp.zeros((_LANES,), jnp.float32) + 1.0,
                jnp.zeros((_LANES,), jnp.float32),
            )
            hit1 = seg_hit(a_i, a_r, jnp.minimum(num_patches, a_r + _LANES))
            hit2 = jnp.logical_and(
                a_r + _LANES > num_patches,
                seg_hit(a_i + 1, 0, a_r + _LANES - num_patches),
            )
            active = jnp.logical_or(hit1, hit2)
            jtile = c // 8

            @pl.when(active)
            def _():
                off = (c - jtile * 8) * _LANES
                handles = [pltpu.async_copy(piece_src(jtile, 0), slabs[0],
                                            sems[0])]
                csum = jnp.zeros((_LANES,), jnp.float32)
                for j in range(pieces):
                    if j + 1 < pieces:
                        handles.append(
                            pltpu.async_copy(piece_src(jtile, j + 1),
                                             slabs[(j + 1) % 2],
                                             sems[(j + 1) % 2])
                        )
                    handles[j].wait()
                    csum = csum + reduce_half(slabs[j % 2], off)
                acc_v[...] = acc_v[...] + csum * m

            pj2 = pj + _LANES
            wrap = pj2 >= num_patches
            pj2 = jnp.where(wrap, pj2 - num_patches, pj2)
            pi2 = jnp.where(wrap, pi + 1, pi)
            a_r2 = a_r + _LANES
            awrap = a_r2 >= num_patches
            a_r2 = jnp.where(awrap, a_r2 - num_patches, a_r2)
            a_i2 = jnp.where(awrap, a_i + 1, a_i)
            return (pi2, pj2, a_i2, a_r2)

        lax.fori_loop(
            0, full_chunks, chunk_body,
            (jnp.zeros((_LANES,), jnp.int32), lane,
             jnp.zeros((), jnp.int32), jnp.zeros((), jnp.int32)),
        )

        pltpu.sync_copy(acc_v, out_hbm.at[wid])

        @pl.when(wid == 0)
        def _():
            count = jnp.maximum(0, i_hi - i_lo) * jnp.maximum(0, j_hi - j_lo)
            acc_v[...] = (jnp.zeros((_LANES,), jnp.int32) + count).astype(
                jnp.float32
            )
            pltpu.sync_copy(acc_v, cnt_hbm)

    return sc_reduce


def kernel(attn_weights, bbox):
    num_patches = SEARCH_SIZE // PATCH_SIZE
    search_seq_len = num_patches**2

    num_layers, batch, heads, seq, seq2 = attn_weights.shape
    assert seq == search_seq_len and seq2 == search_seq_len
    num_mats = num_layers * batch * heads
    rows_total = num_mats * seq

    attn3 = attn_weights.reshape(num_mats, seq, seq)
    bbox_pad = jnp.zeros((_LANES,), jnp.int32).at[:4].set(bbox)

    partials, cntv = _make_sc_reduce(num_mats, seq, num_patches)(
        attn3, bbox_pad
    )
    total = jnp.sum(partials)
    count = cntv[0]
    safe_count = jnp.maximum(count, jnp.float32(1.0))

    x, y, w, h = bbox[0], bbox[1], bbox[2], bbox[3]
    i_hi = jnp.minimum(num_patches, (y + h + PATCH_SIZE - 1) // PATCH_SIZE)
    j_hi = jnp.minimum(num_patches, (x + w + PATCH_SIZE - 1) // PATCH_SIZE)
    tail0 = (seq // 128) * 128
    max_col = (i_hi - 1) * num_patches + (j_hi - 1)
    has_tail = jnp.logical_and(count > 0, max_col >= tail0)

    def _tail_sum(_):
        i_lo = jnp.maximum(0, y // PATCH_SIZE)
        j_lo = jnp.maximum(0, x // PATCH_SIZE)
        p = jnp.arange(tail0, search_seq_len, dtype=jnp.int32)
        pi = p // num_patches
        pj = p % num_patches
        tail_mask = (
            (pi >= i_lo) & (pi < i_hi) & (pj >= j_lo) & (pj < j_hi)
        ).astype(jnp.float32)
        return jnp.einsum(
            "mrk,k->", attn3[:, :, tail0:], tail_mask,
            preferred_element_type=jnp.float32,
        )

    tail_total = lax.cond(has_tail, _tail_sum, lambda _: jnp.float32(0.0), 0)

    loss = (total + tail_total) / (safe_count * jnp.float32(rows_total))
    return jnp.where(count == 0, jnp.float32(0.0), loss)
